# baseline (speedup 1.0000x reference)
import functools

import jax
import jax.numpy as jnp
from jax import lax
from jax.experimental import pallas as pl
from jax.experimental.pallas import tpu as pltpu
from jax.experimental.pallas import tpu_sc as plsc

_POS_OFFSET = 2


@functools.lru_cache(maxsize=None)
def _make_sc_lookup(B, S, D, dtype):
    info = plsc.get_sparse_core_info()
    num_workers = info.num_cores * info.num_subcores
    L = info.num_lanes
    rows_per_w = S // num_workers
    rb = 2 * L
    n_chunks = rows_per_w // rb
    mesh = plsc.VectorSubcoreMesh(core_axis_name="c", subcore_axis_name="s")

    nbuf = 3

    def body(table_hbm, out_hbm, *scratch):
        bufs = scratch[:nbuf]
        idxs = scratch[nbuf:2 * nbuf]
        rsems = scratch[2 * nbuf:3 * nbuf]
        wsems = scratch[3 * nbuf:4 * nbuf]
        wid = lax.axis_index("s") * info.num_cores + lax.axis_index("c")
        base = wid * rows_per_w
        pending_writes = {p: [] for p in range(nbuf)}
        reads = {}
        lane = lax.iota(jnp.int32, L)

        def start_read(j):
            p = j % nbuf
            r2 = base + j * rb + _POS_OFFSET
            for v in range(rb // L):
                idxs[p][pl.ds(v * L, L)] = lane + (r2 + v * L)
            reads[j] = pltpu.async_copy(
                table_hbm.at[idxs[p]], bufs[p], rsems[p])

        for j in range(min(nbuf, n_chunks)):
            start_read(j)
        for j in range(n_chunks):
            p = j % nbuf
            reads[j].wait()
            r0 = base + j * rb
            for b in range(B):
                pending_writes[p].append(pltpu.async_copy(
                    bufs[p], out_hbm.at[b, pl.ds(r0, rb), :], wsems[p]))
            nxt = j + nbuf
            if nxt < n_chunks:
                q = nxt % nbuf
                for w in pending_writes[q]:
                    w.wait()
                pending_writes[q] = []
                start_read(nxt)
        for p in range(nbuf):
            for w in pending_writes[p]:
                w.wait()

    return pl.kernel(
        body,
        out_type=jax.ShapeDtypeStruct((B, S, D), dtype),
        mesh=mesh,
        scratch_types=(
            [pltpu.VMEM((rb, D), dtype)] * nbuf
            + [pltpu.VMEM((rb,), jnp.int32)] * nbuf
            + [pltpu.SemaphoreType.DMA] * (2 * nbuf)
        ),
    )


@functools.lru_cache(maxsize=None)
def _make_tc_broadcast(Bt, S, D, dtype, bs=512):
    nb = S // bs

    def tc_body(a_ref, h_ref, o_ref):
        a = a_ref[...]
        h = h_ref[...]
        o_ref[0] = jnp.concatenate([a[_POS_OFFSET:, :], h[:_POS_OFFSET, :]],
                                   axis=0)

    return pl.pallas_call(
        tc_body,
        grid=(nb, Bt),
        in_specs=[
            pl.BlockSpec((bs, D), lambda i, b: (i, 0)),
            pl.BlockSpec((8, D), lambda i, b: ((i + 1) * (bs // 8), 0)),
        ],
        out_specs=pl.BlockSpec((1, bs, D), lambda i, b: (b, i, 0)),
        out_shape=jax.ShapeDtypeStruct((Bt, S, D), dtype),
    )


_SC_BATCHES = 2


@jax.jit
def kernel(inputs_embeds, table):
    B, S, _ = inputs_embeds.shape
    D = table.shape[1]
    k = _SC_BATCHES
    sc_part = _make_sc_lookup(k, S, D, table.dtype)(table)
    tc_part = _make_tc_broadcast(B - k, S, D, table.dtype)(table, table)
    return jnp.concatenate([sc_part, tc_part], axis=0)


# speedup vs baseline: 2.0030x; 2.0030x over previous
import functools

import jax
import jax.numpy as jnp
from jax import lax
from jax.experimental import pallas as pl
from jax.experimental.pallas import tpu as pltpu
from jax.experimental.pallas import tpu_sc as plsc

_POS_OFFSET = 2


@functools.lru_cache(maxsize=None)
def _make_sc_lookup(B, S, D, dtype):
    info = plsc.get_sparse_core_info()
    num_workers = info.num_cores * info.num_subcores
    L = info.num_lanes
    rows_per_w = S // num_workers
    rb = 2 * L
    n_chunks = rows_per_w // rb
    mesh = plsc.VectorSubcoreMesh(core_axis_name="c", subcore_axis_name="s")

    nbuf = 3

    def body(table_hbm, out_hbm, *scratch):
        bufs = scratch[:nbuf]
        idxs = scratch[nbuf:2 * nbuf]
        rsems = scratch[2 * nbuf:3 * nbuf]
        wsems = scratch[3 * nbuf:4 * nbuf]
        wid = lax.axis_index("s") * info.num_cores + lax.axis_index("c")
        base = wid * rows_per_w
        pending_writes = {p: [] for p in range(nbuf)}
        reads = {}
        lane = lax.iota(jnp.int32, L)

        def start_read(j):
            p = j % nbuf
            r2 = base + j * rb + _POS_OFFSET
            for v in range(rb // L):
                idxs[p][pl.ds(v * L, L)] = lane + (r2 + v * L)
            reads[j] = pltpu.async_copy(
                table_hbm.at[idxs[p]], bufs[p], rsems[p])

        for j in range(min(nbuf, n_chunks)):
            start_read(j)
        for j in range(n_chunks):
            p = j % nbuf
            reads[j].wait()
            r0 = base + j * rb
            for b in range(B):
                pending_writes[p].append(pltpu.async_copy(
                    bufs[p], out_hbm.at[b, pl.ds(r0, rb), :], wsems[p]))
            nxt = j + nbuf
            if nxt < n_chunks:
                q = nxt % nbuf
                for w in pending_writes[q]:
                    w.wait()
                pending_writes[q] = []
                start_read(nxt)
        for p in range(nbuf):
            for w in pending_writes[p]:
                w.wait()

    return pl.kernel(
        body,
        out_type=jax.ShapeDtypeStruct((B, S, D), dtype),
        mesh=mesh,
        scratch_types=(
            [pltpu.VMEM((rb, D), dtype)] * nbuf
            + [pltpu.VMEM((rb,), jnp.int32)] * nbuf
            + [pltpu.SemaphoreType.DMA] * (2 * nbuf)
        ),
    )


@jax.jit
def kernel(inputs_embeds, table):
    B, S, _ = inputs_embeds.shape
    D = table.shape[1]
    return _make_sc_lookup(B, S, D, table.dtype)(table)
